# Initial kernel scaffold; baseline (speedup 1.0000x reference)
import jax
import jax.numpy as jnp
from jax.experimental import pallas as pl


def kernel(card_ids, enhancements, editions, seals, rank_emb, suit_emb, enhancement_emb, edition_emb, seal_emb):
    raise NotImplementedError("write your pallas kernel here")


# speedup vs baseline: 26.6465x; 26.6465x over previous
import functools

import jax
import jax.numpy as jnp
from jax import lax
from jax.experimental import pallas as pl
from jax.experimental.pallas import tpu as pltpu
from jax.experimental.pallas import tpu_sc as plsc

NUM_RANKS = 13
NUM_SUITS = 4
NUM_ENHANCEMENTS = 9
NUM_EDITIONS = 4
NUM_SEALS = 5
D = 64
B, N = 16384, 50
NCOMBO = NUM_ENHANCEMENTS * NUM_EDITIONS * NUM_SEALS
NFUSED = 52 * NCOMBO

TOTAL = B * N
NC, NS, L = 2, 16, 16
NW = NC * NS
PER_W = TOTAL // NW
GSUB = 8
CHUNK = GSUB * 128
NCHUNK = PER_W // CHUNK


def _fused_table_body(rank_ref, suit_ref, enh_ref, ed_ref, seal_ref, out_ref):
    row = lax.broadcasted_iota(jnp.int32, (NFUSED, 1), 0)
    c = row // NCOMBO
    j = row % NCOMBO

    def onehot(idx, k):
        cols = lax.broadcasted_iota(jnp.int32, (NFUSED, k), 1)
        return (cols == idx).astype(jnp.float32)

    acc = jnp.dot(onehot(c // 4, NUM_RANKS), rank_ref[...],
                  preferred_element_type=jnp.float32)
    acc += jnp.dot(onehot(c % 4, NUM_SUITS), suit_ref[...],
                   preferred_element_type=jnp.float32)
    acc += jnp.dot(onehot(j // (NUM_EDITIONS * NUM_SEALS), NUM_ENHANCEMENTS),
                   enh_ref[...], preferred_element_type=jnp.float32)
    acc += jnp.dot(onehot((j // NUM_SEALS) % NUM_EDITIONS, NUM_EDITIONS),
                   ed_ref[...], preferred_element_type=jnp.float32)
    acc += jnp.dot(onehot(j % NUM_SEALS, NUM_SEALS), seal_ref[...],
                   preferred_element_type=jnp.float32)
    out_ref[...] = acc


_fused_table = pl.pallas_call(
    _fused_table_body,
    out_shape=jax.ShapeDtypeStruct((NFUSED, D), jnp.float32),
)


_BB = 2048


def _idx_mask_body(card_ref, enh_ref, ed_ref, seal_ref, cidx_ref, mask_ref):
    card = card_ref[...]
    cidx_ref[...] = (card * NCOMBO + enh_ref[...] * (NUM_EDITIONS * NUM_SEALS)
                     + ed_ref[...] * NUM_SEALS + seal_ref[...])
    mask_ref[...] = (card >= 0).astype(jnp.int8)


_idx_mask = pl.pallas_call(
    _idx_mask_body,
    grid=(B // _BB,),
    in_specs=[pl.BlockSpec((_BB, N), lambda i: (i, 0))] * 4,
    out_specs=[pl.BlockSpec((_BB, N), lambda i: (i, 0))] * 2,
    out_shape=[
        jax.ShapeDtypeStruct((B, N), jnp.int32),
        jax.ShapeDtypeStruct((B, N), jnp.int8),
    ],
)


def _sc_gather_body(fused_hbm, cidx_hbm, out_hbm, idx_v, rows_v, sem):
    wid = lax.axis_index("s") * NC + lax.axis_index("c")
    idx_base = wid * (PER_W // 128)
    out_base = wid * PER_W

    def body(i, carry):
        pltpu.sync_copy(cidx_hbm.at[pl.ds(idx_base + i * GSUB, GSUB)], idx_v)
        copies = [
            pltpu.async_copy(fused_hbm.at[idx_v.at[j]],
                             rows_v.at[pl.ds(j * 128, 128)], sem)
            for j in range(GSUB)
        ]
        for cp in copies:
            cp.wait()
        pltpu.sync_copy(rows_v, out_hbm.at[pl.ds(out_base + i * CHUNK, CHUNK)])
        return carry

    lax.fori_loop(0, NCHUNK, body, 0)


_sc_gather = functools.partial(
    pl.kernel,
    out_type=jax.ShapeDtypeStruct((TOTAL, D), jnp.float32),
    mesh=plsc.VectorSubcoreMesh(core_axis_name="c", subcore_axis_name="s",
                                num_cores=NC, num_subcores=NS),
    compiler_params=pltpu.CompilerParams(use_tc_tiling_on_sc=False),
    scratch_types=[
        pltpu.VMEM((GSUB, 128), jnp.int32),
        pltpu.VMEM((CHUNK, D), jnp.float32),
        pltpu.SemaphoreType.DMA,
    ],
)(_sc_gather_body)


def kernel(card_ids, enhancements, editions, seals, rank_emb, suit_emb,
           enhancement_emb, edition_emb, seal_emb):
    card_ids = card_ids.astype(jnp.int32)
    enhancements = enhancements.astype(jnp.int32)
    editions = editions.astype(jnp.int32)
    seals = seals.astype(jnp.int32)

    fused = _fused_table(rank_emb, suit_emb, enhancement_emb, edition_emb,
                         seal_emb)
    cidx, mask8 = _idx_mask(card_ids, enhancements, editions, seals)
    toks = _sc_gather(fused, cidx.reshape(TOTAL // 128, 128))
    return toks.reshape(B, N, D), mask8.astype(jnp.bool_)
